# baseline (speedup 1.0000x reference)
import functools

import jax
import jax.numpy as jnp
from jax import lax
from jax.experimental import pallas as pl
from jax.experimental.pallas import tpu as pltpu
from jax.experimental.pallas import tpu_sc as plsc

B = 16384
D = 128
CHANS = (96, 192, 384, 768)

_NC, _NS = 2, 16
_NW = _NC * _NS
_NCHUNK = 2
_H = B // _NCHUNK
_BPW = _H // _NW


def _gather_body(table_hbm, idx_hbm, out_hbm, idx_v, rows_v, sem, *, chunk):
    wid = lax.axis_index("s") * _NC + lax.axis_index("c")
    base = chunk * _H + wid * _BPW
    pltpu.sync_copy(idx_hbm.at[pl.ds(base, _BPW)], idx_v)
    pltpu.async_copy(table_hbm.at[idx_v], rows_v, sem).wait()
    pltpu.sync_copy(rows_v, out_hbm.at[pl.ds(wid * _BPW, _BPW)])


@functools.partial(jax.jit, static_argnums=2)
def _sc_gather_chunk(table, idx, chunk):
    mesh = plsc.VectorSubcoreMesh(core_axis_name="c", subcore_axis_name="s")
    return pl.kernel(
        functools.partial(_gather_body, chunk=chunk),
        mesh=mesh,
        out_type=jax.ShapeDtypeStruct((_H, D), jnp.float32),
        scratch_types=[
            pltpu.VMEM((_BPW,), jnp.int32),
            pltpu.VMEM((_BPW, D), jnp.float32),
            pltpu.SemaphoreType.DMA,
        ],
    )(table, idx)


_TB = 2048
_BLKS = _H // _TB


def _mm_compute(emb_ref, w0t, w1t, w2, w3, b0, b1, b2, b3, o0t, o1t, o2, o3):
    e = emb_ref[...]
    dn_t = (((1,), (1,)), ((), ()))
    o0t[...] = (
        jax.lax.dot_general(w0t[...], e, dn_t, preferred_element_type=jnp.float32)
        + b0[...]
    )
    o1t[...] = (
        jax.lax.dot_general(w1t[...], e, dn_t, preferred_element_type=jnp.float32)
        + b1[...]
    )
    o2[...] = jnp.dot(e, w2[...], preferred_element_type=jnp.float32) + b2[...]
    o3[...] = jnp.dot(e, w3[...], preferred_element_type=jnp.float32) + b3[...]


def _mm_body_carry(emb_ref, w0t, w1t, w2, w3, b0, b1, b2, b3,
                   p0, p1, p2, p3, o0t, o1t, o2, o3):
    del p0, p1, p2, p3
    _mm_compute(emb_ref, w0t, w1t, w2, w3, b0, b1, b2, b3, o0t, o1t, o2, o3)


_OUT_SHAPE = [
    jax.ShapeDtypeStruct((CHANS[0], B), jnp.float32),
    jax.ShapeDtypeStruct((CHANS[1], B), jnp.float32),
    jax.ShapeDtypeStruct((B, CHANS[2]), jnp.float32),
    jax.ShapeDtypeStruct((B, CHANS[3]), jnp.float32),
]


def _specs(off):
    full = lambda shape: pl.BlockSpec(shape, lambda i: (0,) * len(shape))
    in_specs = [
        pl.BlockSpec((_TB, D), lambda i: (i, 0)),
        full((CHANS[0], D)), full((CHANS[1], D)),
        full((D, CHANS[2])), full((D, CHANS[3])),
        full((CHANS[0], 1)), full((CHANS[1], 1)),
        full((CHANS[2],)), full((CHANS[3],)),
    ]
    out_specs = [
        pl.BlockSpec((CHANS[0], _TB), lambda i: (0, i + off)),
        pl.BlockSpec((CHANS[1], _TB), lambda i: (0, i + off)),
        pl.BlockSpec((_TB, CHANS[2]), lambda i: (i + off, 0)),
        pl.BlockSpec((_TB, CHANS[3]), lambda i: (i + off, 0)),
    ]
    return in_specs, out_specs


@jax.jit
def _tc_adapters_first(emb, W0t, W1t, W2, W3, b0c, b1c, b2, b3):
    in_specs, out_specs = _specs(0)
    return pl.pallas_call(
        _mm_compute,
        grid=(_BLKS,),
        in_specs=in_specs,
        out_specs=out_specs,
        out_shape=_OUT_SHAPE,
    )(emb, W0t, W1t, W2, W3, b0c, b1c, b2, b3)


@jax.jit
def _tc_adapters_second(emb, W0t, W1t, W2, W3, b0c, b1c, b2, b3, *carry):
    in_specs, out_specs = _specs(_BLKS)
    anyspec = pl.BlockSpec(memory_space=pl.ANY)
    return pl.pallas_call(
        _mm_body_carry,
        grid=(_BLKS,),
        in_specs=in_specs + [anyspec] * 4,
        out_specs=out_specs,
        out_shape=_OUT_SHAPE,
        input_output_aliases={9: 0, 10: 1, 11: 2, 12: 3},
    )(emb, W0t, W1t, W2, W3, b0c, b1c, b2, b3, *carry)


def kernel(x, class_ids, table, W0, W1, W2, W3, b0, b1, b2, b3):
    idx = class_ids.astype(jnp.int32)
    W0t, W1t = W0.T, W1.T
    b0c, b1c = b0[:, None], b1[:, None]
    emb0 = _sc_gather_chunk(table, idx, 0)
    emb1 = _sc_gather_chunk(table, idx, 1)
    carry = _tc_adapters_first(emb0, W0t, W1t, W2, W3, b0c, b1c, b2, b3)
    o0t, o1t, o2, o3 = _tc_adapters_second(
        emb1, W0t, W1t, W2, W3, b0c, b1c, b2, b3, *tuple(carry)
    )
    return (o0t.T, o1t.T, o2, o3)


# speedup vs baseline: 1.0493x; 1.0493x over previous
import functools

import jax
import jax.numpy as jnp
from jax import lax
from jax.experimental import pallas as pl
from jax.experimental.pallas import tpu as pltpu
from jax.experimental.pallas import tpu_sc as plsc

B = 16384
D = 128
CHANS = (96, 192, 384, 768)

_NC, _NS = 2, 16
_NW = _NC * _NS
_BPW = B // _NW


def _gather_body(table_hbm, idx_hbm, out_hbm, idx_v, rows_v, sem):
    wid = lax.axis_index("s") * _NC + lax.axis_index("c")
    base = wid * _BPW
    pltpu.sync_copy(idx_hbm.at[pl.ds(base, _BPW)], idx_v)
    pltpu.async_copy(table_hbm.at[idx_v], rows_v, sem).wait()
    pltpu.sync_copy(rows_v, out_hbm.at[pl.ds(base, _BPW)])


@jax.jit
def _sc_gather(table, idx):
    mesh = plsc.VectorSubcoreMesh(core_axis_name="c", subcore_axis_name="s")
    return pl.kernel(
        _gather_body,
        mesh=mesh,
        out_type=jax.ShapeDtypeStruct((B, D), jnp.float32),
        scratch_types=[
            pltpu.VMEM((_BPW,), jnp.int32),
            pltpu.VMEM((_BPW, D), jnp.float32),
            pltpu.SemaphoreType.DMA,
        ],
    )(table, idx)


_TB = 1024


def _mm_body(emb_ref, w0t, w1t, w2, w3, b0, b1, b2, b3, o0t, o1t, o2, o3):
    e = emb_ref[...]
    dn_t = (((1,), (1,)), ((), ()))
    o0t[...] = (
        jax.lax.dot_general(w0t[...], e, dn_t, preferred_element_type=jnp.float32)
        + b0[...]
    )
    o1t[...] = (
        jax.lax.dot_general(w1t[...], e, dn_t, preferred_element_type=jnp.float32)
        + b1[...]
    )
    o2[...] = jnp.dot(e, w2[...], preferred_element_type=jnp.float32) + b2[...]
    o3[...] = jnp.dot(e, w3[...], preferred_element_type=jnp.float32) + b3[...]


@jax.jit
def _tc_adapters(emb, W0t, W1t, W2, W3, b0c, b1c, b2, b3):
    full = lambda shape: pl.BlockSpec(shape, lambda i: (0,) * len(shape))
    return pl.pallas_call(
        _mm_body,
        grid=(B // _TB,),
        in_specs=[
            pl.BlockSpec((_TB, D), lambda i: (i, 0)),
            full((CHANS[0], D)), full((CHANS[1], D)),
            full((D, CHANS[2])), full((D, CHANS[3])),
            full((CHANS[0], 1)), full((CHANS[1], 1)),
            full((CHANS[2],)), full((CHANS[3],)),
        ],
        out_specs=[
            pl.BlockSpec((CHANS[0], _TB), lambda i: (0, i)),
            pl.BlockSpec((CHANS[1], _TB), lambda i: (0, i)),
            pl.BlockSpec((_TB, CHANS[2]), lambda i: (i, 0)),
            pl.BlockSpec((_TB, CHANS[3]), lambda i: (i, 0)),
        ],
        out_shape=[
            jax.ShapeDtypeStruct((CHANS[0], B), jnp.float32),
            jax.ShapeDtypeStruct((CHANS[1], B), jnp.float32),
            jax.ShapeDtypeStruct((B, CHANS[2]), jnp.float32),
            jax.ShapeDtypeStruct((B, CHANS[3]), jnp.float32),
        ],
    )(emb, W0t, W1t, W2, W3, b0c, b1c, b2, b3)


def kernel(x, class_ids, table, W0, W1, W2, W3, b0, b1, b2, b3):
    emb = _sc_gather(table, class_ids.astype(jnp.int32))
    o0t, o1t, o2, o3 = _tc_adapters(
        emb, W0.T, W1.T, W2, W3, b0[:, None], b1[:, None], b2, b3
    )
    return (o0t.T, o1t.T, o2, o3)
